# Initial kernel scaffold; baseline (speedup 1.0000x reference)
import jax
import jax.numpy as jnp
from jax.experimental import pallas as pl


def kernel(x, adj, W0, a0, W1, a1, W_out, a_out):
    raise NotImplementedError("write your pallas kernel here")


# speedup vs baseline: 2.1218x; 2.1218x over previous
import functools

import jax
import jax.numpy as jnp
from jax.experimental import pallas as pl
from jax.experimental.pallas import tpu as pltpu

ALPHA = 0.2
NEG = -9e15

_INTERPRET = False


def _divisor_block(n, target):
    best = None
    for b in range(8, min(n, target) + 1, 8):
        if n % b == 0:
            best = b
    return best if best is not None else n


def _leaky_relu(v):
    return jnp.maximum(v, ALPHA * v)


def _elu(v):
    return jnp.where(v > 0, v, jnp.exp(jnp.minimum(v, 0.0)) - 1.0)


def _attend(mask, s, dt, wh):
    t = _leaky_relu(s + dt)
    e = jnp.where(mask, t, NEG)
    m = jnp.max(e, axis=1, keepdims=True)
    p = jnp.exp(e - m)
    l = jnp.sum(p, axis=1, keepdims=True)
    acc = jnp.dot(p, wh, preferred_element_type=jnp.float32)
    return acc / l


def _proj_body(x_ref, w0_ref, a0_ref, w1_ref, a1_ref,
               wh0_ref, s0_ref, d0_ref, wh1_ref, s1_ref, d1_ref):
    x = x_ref[...]
    d_hid = w0_ref.shape[1]
    for w_ref, a_ref, wh_ref, s_ref, d_ref in (
        (w0_ref, a0_ref, wh0_ref, s0_ref, d0_ref),
        (w1_ref, a1_ref, wh1_ref, s1_ref, d1_ref),
    ):
        wh = jnp.dot(x, w_ref[...], preferred_element_type=jnp.float32)
        wh_ref[...] = wh
        s_ref[...] = jnp.dot(wh, a_ref[:d_hid], preferred_element_type=jnp.float32)
        d_ref[...] = jnp.dot(wh, a_ref[d_hid:], preferred_element_type=jnp.float32)


def _projections(x, w0, a0, w1, a1):
    n, nfeat = x.shape
    d_hid = w0.shape[1]
    br = _divisor_block(n, 2500)
    grid = (n // br,)
    out_shapes = []
    for _ in range(2):
        out_shapes += [
            jax.ShapeDtypeStruct((n, d_hid), jnp.float32),
            jax.ShapeDtypeStruct((n, 1), jnp.float32),
            jax.ShapeDtypeStruct((n, 1), jnp.float32),
        ]
    full = lambda shape: pl.BlockSpec(shape, lambda i: (0, 0))
    row = lambda width: pl.BlockSpec((br, width), lambda i: (i, 0))
    return pl.pallas_call(
        _proj_body,
        grid=grid,
        in_specs=[
            row(nfeat),
            full(w0.shape), full(a0.shape),
            full(w1.shape), full(a1.shape),
        ],
        out_specs=[row(d_hid), row(1), row(1)] * 2,
        out_shape=out_shapes,
        interpret=_INTERPRET,
    )(x, w0, a0, w1, a1)


def _flash12_body(adj_ref, s0_ref, d0t_ref, wh0_ref, s1_ref, d1t_ref, wh1_ref,
                  wo_ref, ao_ref, who_ref, s3_ref, d3_ref, *, d_hid, n_cls):
    mask = adj_ref[...] > 0.0
    h0 = _elu(_attend(mask, s0_ref[...], d0t_ref[...], wh0_ref[...]))
    h1 = _elu(_attend(mask, s1_ref[...], d1t_ref[...], wh1_ref[...]))
    who = (jnp.dot(h0, wo_ref[:d_hid], preferred_element_type=jnp.float32)
           + jnp.dot(h1, wo_ref[d_hid:], preferred_element_type=jnp.float32))
    who_ref[...] = who
    s3_ref[...] = jnp.dot(who, ao_ref[:n_cls], preferred_element_type=jnp.float32)
    d3_ref[...] = jnp.dot(who, ao_ref[n_cls:], preferred_element_type=jnp.float32)


def _flash12(adj, s0, d0t, wh0, s1, d1t, wh1, wo, ao, br):
    n = adj.shape[0]
    d_hid = wh0.shape[1]
    n_cls = wo.shape[1]
    grid = (n // br,)
    full = lambda shape: pl.BlockSpec(shape, lambda i: (0, 0))
    rowblk = lambda width: pl.BlockSpec((br, width), lambda i: (i, 0))
    body = functools.partial(_flash12_body, d_hid=d_hid, n_cls=n_cls)
    return pl.pallas_call(
        body,
        grid=grid,
        in_specs=[
            rowblk(n),
            rowblk(1),
            full(d0t.shape),
            full(wh0.shape),
            rowblk(1),
            full(d1t.shape),
            full(wh1.shape),
            full(wo.shape), full(ao.shape),
        ],
        out_specs=[rowblk(n_cls), rowblk(1), rowblk(1)],
        out_shape=[
            jax.ShapeDtypeStruct((n, n_cls), jnp.float32),
            jax.ShapeDtypeStruct((n, 1), jnp.float32),
            jax.ShapeDtypeStruct((n, 1), jnp.float32),
        ],
        interpret=_INTERPRET,
    )(adj, s0, d0t, wh0, s1, d1t, wh1, wo, ao)


def _flash3_body(adj_ref, s_ref, dt_ref, wh_ref, out_ref):
    mask = adj_ref[...] > 0.0
    out_ref[...] = _elu(_attend(mask, s_ref[...], dt_ref[...], wh_ref[...]))


def _flash3(adj, s3, d3t, who, br):
    n = adj.shape[0]
    n_cls = who.shape[1]
    grid = (n // br,)
    return pl.pallas_call(
        _flash3_body,
        grid=grid,
        in_specs=[
            pl.BlockSpec((br, n), lambda i: (i, 0)),
            pl.BlockSpec((br, 1), lambda i: (i, 0)),
            pl.BlockSpec(d3t.shape, lambda i: (0, 0)),
            pl.BlockSpec(who.shape, lambda i: (0, 0)),
        ],
        out_specs=pl.BlockSpec((br, n_cls), lambda i: (i, 0)),
        out_shape=jax.ShapeDtypeStruct((n, n_cls), jnp.float32),
        interpret=_INTERPRET,
    )(adj, s3, d3t, who)


def kernel(x, adj, W0, a0, W1, a1, W_out, a_out):
    n = x.shape[0]
    br = _divisor_block(n, 200)
    wh0, s0, d0, wh1, s1, d1 = _projections(x, W0, a0, W1, a1)
    d0t = jnp.reshape(d0, (1, n))
    d1t = jnp.reshape(d1, (1, n))
    who, s3, d3 = _flash12(adj, s0, d0t, wh0, s1, d1t, wh1, W_out, a_out, br)
    d3t = jnp.reshape(d3, (1, n))
    return _flash3(adj, s3, d3t, who, br)
